# baseline (speedup 1.0000x reference)
import functools

import jax
import jax.numpy as jnp
from jax import lax
from jax.experimental import pallas as pl
from jax.experimental.pallas import tpu as pltpu
from jax.experimental.pallas import tpu_sc as plsc

_N = 32768
_D = 512
_S = 16
_L = 16

_TC_ROWS = 24576
_TC_BLK = 4096

_SC_ROWS = _N - _TC_ROWS
_CB = 128
_NJ = _CB // _L
_RSL = _SC_ROWS // 8
_R = 512
_NBLK = _RSL // _R


def _tc_body(ids_ref, x_ref, o_ref, starts_ref):
    i = pl.program_id(0)

    @pl.when(i == 0)
    def _init():
        o_ref[...] = jnp.full_like(o_ref, -jnp.inf)
        ids = ids_ref[...]
        starts_ref[0] = 0
        for s in range(1, _S):
            starts_ref[s] = jnp.sum((ids < s).astype(jnp.int32))
        starts_ref[_S] = _N

    x = x_ref[...]
    base = i * _TC_BLK
    rowid = base + jax.lax.broadcasted_iota(jnp.int32, (_TC_BLK, 1), 0)
    for s in range(_S):
        lo = starts_ref[s]
        hi = starts_ref[s + 1]
        present = (hi > base) & (lo < base + _TC_BLK)
        whole = (lo <= base) & (hi >= base + _TC_BLK)

        @pl.when(present & whole)
        def _plain(s=s):
            col = jnp.max(x, axis=0)
            o_ref[s, :] = jnp.maximum(o_ref[s, :], col)

        @pl.when(present & ~whole)
        def _masked(s=s, lo=lo, hi=hi):
            m = (rowid >= lo) & (rowid < hi)
            col = jnp.max(jnp.where(m, x, -jnp.inf), axis=0)
            o_ref[s, :] = jnp.maximum(o_ref[s, :], col)


def _tc_partial(features, ids_rs):
    nblk = _TC_ROWS // _TC_BLK
    return pl.pallas_call(
        _tc_body,
        grid=(nblk,),
        in_specs=[
            pl.BlockSpec((_N // 128, 128), lambda i: (0, 0)),
            pl.BlockSpec((_TC_BLK, _D), lambda i: (i, 0)),
        ],
        out_specs=pl.BlockSpec((_S, _D), lambda i: (0, 0)),
        out_shape=jax.ShapeDtypeStruct((_S, _D), jnp.float32),
        scratch_shapes=[pltpu.SMEM((_S + 1,), jnp.int32)],
    )(ids_rs, features)


def _sc_body(feat_hbm, ids_hbm, out_hbm, buf0_v, ids_v, acc_v, tmp_v,
             part_sh, sem0):
    c = lax.axis_index("c")
    sub = lax.axis_index("s")
    cb_local = sub // 8
    rs = sub % 8
    col0 = (c * 2 + cb_local) * _CB
    row0 = _TC_ROWS + rs * _RSL

    pltpu.sync_copy(ids_hbm.at[pl.ds(row0, _RSL)], ids_v)

    targets = lax.iota(jnp.int32, _L)
    lo0 = jnp.zeros((_L,), jnp.int32)
    hi0 = jnp.full((_L,), _RSL, jnp.int32)

    def bs_body(_, carry):
        lo, hi = carry
        mid = lax.shift_right_logical(lo + hi, 1)
        vals = plsc.load_gather(ids_v, [mid])
        pred = vals < targets
        return jnp.where(pred, mid + 1, lo), jnp.where(pred, hi, mid)

    lo0, hi0 = lax.fori_loop(0, 12, bs_body, (lo0, hi0))
    ends0 = jnp.where(
        targets == _S - 1, jnp.int32(_RSL),
        lo0.at[jnp.minimum(targets + 1, _S - 1)].get(mode="promise_in_bounds"))

    minus_inf = jnp.full((_L,), -jnp.inf, jnp.float32)
    for s in range(_S):
        for j in range(_NJ):
            acc_v[s, pl.ds(j * _L, _L)] = minus_inf

    def _start(b, buf, sem):
        pltpu.async_copy(
            feat_hbm.at[pl.ds(row0 + b * _R, _R), pl.ds(col0, _CB)], buf, sem)

    def _wait(b, buf, sem):
        pltpu.make_async_copy(
            feat_hbm.at[pl.ds(row0 + b * _R, _R), pl.ds(col0, _CB)], buf,
            sem).wait()

    zero16 = jnp.zeros((_L,), jnp.int32)

    def _process(buf, blk_lo):
        def seg_body(s, carry):
            splat = zero16 + s
            st = lo0.at[splat].get(mode="promise_in_bounds")[0]
            en = ends0.at[splat].get(mode="promise_in_bounds")[0]
            lo_b = jnp.maximum(st, blk_lo) - blk_lo
            hi_b = jnp.minimum(en, blk_lo + _R) - blk_lo

            @pl.when(hi_b > lo_b)
            def _run():
                accs0 = tuple(
                    acc_v[s, pl.ds(j * _L, _L)] for j in range(_NJ))

                def row_body(r, accs_in):
                    return tuple(
                        jnp.maximum(accs_in[j], buf[r, pl.ds(j * _L, _L)])
                        for j in range(_NJ))

                accs = plsc.parallel_loop(
                    lo_b, hi_b, unroll=1, carry=accs0)(row_body)

                for j in range(_NJ):
                    acc_v[s, pl.ds(j * _L, _L)] = accs[j]
            return carry

        lax.fori_loop(0, _S, seg_body, 0)

    def blk_body(b, carry):
        _start(b, buf0_v, sem0)
        _wait(b, buf0_v, sem0)
        _process(buf0_v, b * _R)
        return carry

    lax.fori_loop(0, _NBLK, blk_body, 0)

    pltpu.sync_copy(acc_v, part_sh.at[cb_local, rs])
    plsc.subcore_barrier()

    @pl.when(rs == 0)
    def _combine():
        def fold_body(k, carry):
            pltpu.sync_copy(part_sh.at[cb_local, k], tmp_v)
            for s in range(_S):
                for j in range(_NJ):
                    sl = pl.ds(j * _L, _L)
                    acc_v[s, sl] = jnp.maximum(acc_v[s, sl], tmp_v[s, sl])
            return carry

        lax.fori_loop(1, 8, fold_body, 0)
        pltpu.sync_copy(acc_v, out_hbm.at[:, pl.ds(col0, _CB)])


def _sc_partial(features, ids):
    sc_kernel = functools.partial(
        pl.kernel,
        mesh=plsc.VectorSubcoreMesh(core_axis_name="c", subcore_axis_name="s"),
        compiler_params=pltpu.CompilerParams(needs_layout_passes=False),
        out_type=jax.ShapeDtypeStruct((_S, _D), jnp.float32),
        scratch_types=[
            pltpu.VMEM((_R, _CB), jnp.float32),
            pltpu.VMEM((_RSL,), jnp.int32),
            pltpu.VMEM((_S, _CB), jnp.float32),
            pltpu.VMEM((_S, _CB), jnp.float32),
            pltpu.VMEM_SHARED((2, 8, _S, _CB), jnp.float32),
            pltpu.SemaphoreType.DMA,
        ],
    )(_sc_body)
    return sc_kernel(features, ids)


def _comb_body(a_ref, b_ref, o_ref):
    o_ref[...] = jnp.maximum(a_ref[...], b_ref[...])


def _combine(a, b):
    return pl.pallas_call(
        _comb_body,
        out_shape=jax.ShapeDtypeStruct((_S, _D), jnp.float32),
    )(a, b)


def kernel(features, batch_ids):
    ids32 = batch_ids.astype(jnp.int32)
    tc_part = _tc_partial(features, ids32.reshape(_N // 128, 128))
    sc_part = _sc_partial(features, ids32)
    return _combine(tc_part, sc_part)


# speedup vs baseline: 1.0217x; 1.0217x over previous
import functools

import jax
import jax.numpy as jnp
from jax import lax
from jax.experimental import pallas as pl
from jax.experimental.pallas import tpu as pltpu
from jax.experimental.pallas import tpu_sc as plsc

_N = 32768
_D = 512
_S = 16
_L = 16

_TC_ROWS = 24576
_TC_BLK = 4096

_SC_ROWS = _N - _TC_ROWS
_CB = 128
_NJ = _CB // _L
_RSL = _SC_ROWS // 8
_R = 512
_NBLK = _RSL // _R


def _tc_body(ids_ref, x_ref, o_ref, starts_ref):
    i = pl.program_id(0)

    @pl.when(i == 0)
    def _init():
        o_ref[...] = jnp.full_like(o_ref, -jnp.inf)
        ids = ids_ref[...]
        starts_ref[0] = 0
        for s in range(1, _S):
            starts_ref[s] = jnp.sum((ids < s).astype(jnp.int32))
        starts_ref[_S] = _N

    x = x_ref[...]
    base = i * _TC_BLK
    rowid = base + jax.lax.broadcasted_iota(jnp.int32, (_TC_BLK, 1), 0)
    for s in range(_S):
        lo = starts_ref[s]
        hi = starts_ref[s + 1]
        present = (hi > base) & (lo < base + _TC_BLK)
        whole = (lo <= base) & (hi >= base + _TC_BLK)

        @pl.when(present & whole)
        def _plain(s=s):
            col = jnp.max(x, axis=0)
            o_ref[s, :] = jnp.maximum(o_ref[s, :], col)

        @pl.when(present & ~whole)
        def _masked(s=s, lo=lo, hi=hi):
            m = (rowid >= lo) & (rowid < hi)
            col = jnp.max(jnp.where(m, x, -jnp.inf), axis=0)
            o_ref[s, :] = jnp.maximum(o_ref[s, :], col)


def _tc_partial(features, ids_rs):
    nblk = _TC_ROWS // _TC_BLK
    return pl.pallas_call(
        _tc_body,
        grid=(nblk,),
        in_specs=[
            pl.BlockSpec((_N // 128, 128), lambda i: (0, 0)),
            pl.BlockSpec((_TC_BLK, _D), lambda i: (i, 0)),
        ],
        out_specs=pl.BlockSpec((_S, _D), lambda i: (0, 0)),
        out_shape=jax.ShapeDtypeStruct((_S, _D), jnp.float32),
        scratch_shapes=[pltpu.SMEM((_S + 1,), jnp.int32)],
    )(ids_rs, features)


def _sc_body(feat_hbm, ids_hbm, out_hbm, buf0_v, ids_v, acc_v, tmp_v,
             part_sh, sem0):
    c = lax.axis_index("c")
    sub = lax.axis_index("s")
    cb_local = sub // 8
    rs = sub % 8
    col0 = (c * 2 + cb_local) * _CB
    row0 = _TC_ROWS + rs * _RSL

    pltpu.sync_copy(ids_hbm.at[pl.ds(row0, _RSL)], ids_v)

    targets = lax.iota(jnp.int32, _L)
    lo0 = jnp.zeros((_L,), jnp.int32)
    hi0 = jnp.full((_L,), _RSL, jnp.int32)

    def bs_body(_, carry):
        lo, hi = carry
        mid = lax.shift_right_logical(lo + hi, 1)
        vals = plsc.load_gather(ids_v, [mid])
        pred = vals < targets
        return jnp.where(pred, mid + 1, lo), jnp.where(pred, hi, mid)

    lo0, hi0 = lax.fori_loop(0, 12, bs_body, (lo0, hi0))

    minus_inf = jnp.full((_L,), -jnp.inf, jnp.float32)
    for s in range(_S):
        for j in range(_NJ):
            acc_v[s, pl.ds(j * _L, _L)] = minus_inf

    def _start(b, buf, sem):
        pltpu.async_copy(
            feat_hbm.at[pl.ds(row0 + b * _R, _R), pl.ds(col0, _CB)], buf, sem)

    def _wait(b, buf, sem):
        pltpu.make_async_copy(
            feat_hbm.at[pl.ds(row0 + b * _R, _R), pl.ds(col0, _CB)], buf,
            sem).wait()

    starts = [lo0[s] for s in range(_S)] + [jnp.int32(_RSL)]

    def _process(buf, blk_lo):
        for s in range(_S):
            lo_b = jnp.maximum(starts[s], blk_lo) - blk_lo
            hi_b = jnp.minimum(starts[s + 1], blk_lo + _R) - blk_lo

            @pl.when(hi_b > lo_b)
            def _run(s=s, lo_b=lo_b, hi_b=hi_b):
                accs0 = tuple(
                    acc_v[s, pl.ds(j * _L, _L)] for j in range(_NJ))

                def row_body(r, accs_in):
                    return tuple(
                        jnp.maximum(accs_in[j], buf[r, pl.ds(j * _L, _L)])
                        for j in range(_NJ))

                accs = plsc.parallel_loop(
                    lo_b, hi_b, unroll=1, carry=accs0)(row_body)

                for j in range(_NJ):
                    acc_v[s, pl.ds(j * _L, _L)] = accs[j]

    def blk_body(b, carry):
        _start(b, buf0_v, sem0)
        _wait(b, buf0_v, sem0)
        _process(buf0_v, b * _R)
        return carry

    lax.fori_loop(0, _NBLK, blk_body, 0)

    pltpu.sync_copy(acc_v, part_sh.at[cb_local, rs])
    plsc.subcore_barrier()

    @pl.when(rs == 0)
    def _combine():
        def fold_body(k, carry):
            pltpu.sync_copy(part_sh.at[cb_local, k], tmp_v)
            for s in range(_S):
                for j in range(_NJ):
                    sl = pl.ds(j * _L, _L)
                    acc_v[s, sl] = jnp.maximum(acc_v[s, sl], tmp_v[s, sl])
            return carry

        lax.fori_loop(1, 8, fold_body, 0)
        pltpu.sync_copy(acc_v, out_hbm.at[:, pl.ds(col0, _CB)])


def _sc_partial(features, ids):
    sc_kernel = functools.partial(
        pl.kernel,
        mesh=plsc.VectorSubcoreMesh(core_axis_name="c", subcore_axis_name="s"),
        compiler_params=pltpu.CompilerParams(needs_layout_passes=False),
        out_type=jax.ShapeDtypeStruct((_S, _D), jnp.float32),
        scratch_types=[
            pltpu.VMEM((_R, _CB), jnp.float32),
            pltpu.VMEM((_RSL,), jnp.int32),
            pltpu.VMEM((_S, _CB), jnp.float32),
            pltpu.VMEM((_S, _CB), jnp.float32),
            pltpu.VMEM_SHARED((2, 8, _S, _CB), jnp.float32),
            pltpu.SemaphoreType.DMA,
        ],
    )(_sc_body)
    return sc_kernel(features, ids)


def _comb_body(a_ref, b_ref, o_ref):
    o_ref[...] = jnp.maximum(a_ref[...], b_ref[...])


def _combine(a, b):
    return pl.pallas_call(
        _comb_body,
        out_shape=jax.ShapeDtypeStruct((_S, _D), jnp.float32),
    )(a, b)


def kernel(features, batch_ids):
    ids32 = batch_ids.astype(jnp.int32)
    tc_part = _tc_partial(features, ids32.reshape(_N // 128, 128))
    sc_part = _sc_partial(features, ids32)
    return _combine(tc_part, sc_part)
